# baseline (speedup 1.0000x reference)
import jax
import jax.numpy as jnp
from jax.experimental import pallas as pl

_NG = 128
_GS = 32


def _fps(xyz, K):
    B, N, _ = xyz.shape
    bidx = jnp.arange(B)
    farthest = jnp.zeros((B,), dtype=jnp.int32)
    distances = jnp.full((B, N), 1e10, dtype=jnp.float32)
    center_idx_list = []
    for i in range(K):
        center_idx_list.append(farthest)
        centroid = xyz[bidx, farthest, :][:, None, :]
        dist = jnp.sum((xyz - centroid) ** 2, axis=-1)
        distances = jnp.minimum(distances, dist)
        farthest = jnp.argmax(distances, axis=1).astype(jnp.int32)
    center_idx = jnp.stack(center_idx_list, axis=1)
    centers = xyz[bidx[:, None], center_idx]
    return centers, center_idx


def _cdist(a, b):
    aa = jnp.sum(a * a, axis=-1)[:, :, None]
    bb = jnp.sum(b * b, axis=-1)[:, None, :]
    ab = jnp.einsum('bgd,bnd->bgn', a, b)
    d2 = jnp.maximum(aa + bb - 2.0 * ab, 0.0)
    return jnp.sqrt(d2)


def _sub_kernel(n_ref, c_ref, o_ref):
    o_ref[...] = n_ref[...] - c_ref[...]


def kernel(xyz):
    B, N, _ = xyz.shape
    centers, _ = _fps(xyz, _NG)
    dist = _cdist(centers, xyz)
    _, idx = jax.lax.top_k(-dist, _GS)
    bidx = jnp.arange(B)[:, None, None]
    neighborhood = xyz[bidx, idx]
    n2 = neighborhood.reshape(B * _NG, _GS * 3)
    c2 = jnp.broadcast_to(centers.reshape(B * _NG, 1, 3),
                          (B * _NG, _GS, 3)).reshape(B * _NG, _GS * 3)
    nb = pl.pallas_call(
        _sub_kernel,
        out_shape=jax.ShapeDtypeStruct((B * _NG, _GS * 3), jnp.float32),
    )(n2, c2)
    return nb.reshape(B, _NG, _GS, 3), centers, idx


# speedup vs baseline: 5.1038x; 5.1038x over previous
import jax
import jax.numpy as jnp
from jax.experimental import pallas as pl
from jax.experimental.pallas import tpu as pltpu

_NG = 128
_GS = 32
_B = 16
_N = 8192
_SL = 64
_LN = 128


def _fps_kernel(xs_ref, ys_ref, zs_ref, cent_ref, dist_scr):
    xs = xs_ref[...]
    ys = ys_ref[...]
    zs = zs_ref[...]
    si = jax.lax.broadcasted_iota(jnp.int32, (1, _SL, _LN), 1)
    li = jax.lax.broadcasted_iota(jnp.int32, (1, _SL, _LN), 2)
    fi = si * _LN + li
    dist_scr[...] = jnp.full((_B, _SL, _LN), 1e10, jnp.float32)

    def body(i, far):
        mask = fi == far
        cx = jnp.sum(jnp.where(mask, xs, 0.0), axis=(1, 2), keepdims=True)
        cy = jnp.sum(jnp.where(mask, ys, 0.0), axis=(1, 2), keepdims=True)
        cz = jnp.sum(jnp.where(mask, zs, 0.0), axis=(1, 2), keepdims=True)
        row = jnp.concatenate([cx[:, 0, :], cy[:, 0, :], cz[:, 0, :]],
                              axis=-1)
        cent_ref[:, pl.ds(i, 1), :] = row[:, None, :]
        dx = xs - cx
        dy = ys - cy
        dz = zs - cz
        d = (dx * dx + dy * dy) + dz * dz
        dmin = jnp.minimum(dist_scr[...], d)
        dist_scr[...] = dmin
        m = jnp.max(dmin, axis=(1, 2), keepdims=True)
        far2 = jnp.min(jnp.where(dmin == m, fi, _N), axis=(1, 2),
                       keepdims=True)
        return far2

    jax.lax.fori_loop(0, _NG, body, jnp.zeros((_B, 1, 1), jnp.int32))


def _knn_kernel(xs_ref, ys_ref, zs_ref, cent_ref, idx_ref, d_scr):
    px = xs_ref[0]
    py = ys_ref[0]
    pz = zs_ref[0]
    cg = cent_ref[0]
    cgx = cg[:, 0:1]
    cgy = cg[:, 1:2]
    cgz = cg[:, 2:3]
    aa = (cgx * cgx + cgy * cgy) + cgz * cgz
    bb = (px * px + py * py) + pz * pz
    cbx = cgx.astype(jnp.bfloat16).astype(jnp.float32)
    cby = cgy.astype(jnp.bfloat16).astype(jnp.float32)
    cbz = cgz.astype(jnp.bfloat16).astype(jnp.float32)
    pbx = px.astype(jnp.bfloat16).astype(jnp.float32)
    pby = py.astype(jnp.bfloat16).astype(jnp.float32)
    pbz = pz.astype(jnp.bfloat16).astype(jnp.float32)
    ab = (cbx * pbx + cby * pby) + cbz * pbz
    d2 = jnp.maximum(aa + bb - 2.0 * ab, 0.0)
    d_scr[...] = jnp.sqrt(d2)
    li = jax.lax.broadcasted_iota(jnp.int32, (1, _N), 1)
    for k in range(_GS):
        dcur = d_scr[...]
        m = jnp.min(dcur, axis=1, keepdims=True)
        a = jnp.min(jnp.where(dcur == m, li, _N), axis=1, keepdims=True)
        idx_ref[0, :, k:k + 1] = a
        d_scr[...] = jnp.where(li == a, jnp.inf, dcur)


def kernel(xyz):
    B, N, _ = xyz.shape
    xs = xyz[:, :, 0]
    ys = xyz[:, :, 1]
    zs = xyz[:, :, 2]
    xs3 = xs.reshape(B, _SL, _LN)
    ys3 = ys.reshape(B, _SL, _LN)
    zs3 = zs.reshape(B, _SL, _LN)

    centers = pl.pallas_call(
        _fps_kernel,
        out_shape=jax.ShapeDtypeStruct((B, _NG, 3), jnp.float32),
        scratch_shapes=[pltpu.VMEM((_B, _SL, _LN), jnp.float32)],
    )(xs3, ys3, zs3)

    idx = pl.pallas_call(
        _knn_kernel,
        grid=(B,),
        in_specs=[
            pl.BlockSpec((1, 1, N), lambda b: (b, 0, 0)),
            pl.BlockSpec((1, 1, N), lambda b: (b, 0, 0)),
            pl.BlockSpec((1, 1, N), lambda b: (b, 0, 0)),
            pl.BlockSpec((1, _NG, 3), lambda b: (b, 0, 0)),
        ],
        out_specs=pl.BlockSpec((1, _NG, _GS), lambda b: (b, 0, 0)),
        out_shape=jax.ShapeDtypeStruct((B, _NG, _GS), jnp.int32),
        scratch_shapes=[pltpu.VMEM((_NG, _N), jnp.float32)],
    )(xs[:, None, :], ys[:, None, :], zs[:, None, :], centers)

    bidx = jnp.arange(B)[:, None, None]
    neighborhood = xyz[bidx, idx] - centers[:, :, None, :]
    return neighborhood, centers, idx
